# Initial kernel scaffold; baseline (speedup 1.0000x reference)
import jax
import jax.numpy as jnp
from jax.experimental import pallas as pl


def kernel(inputs, cond_pos, pos_table, cond_table):
    raise NotImplementedError("write your pallas kernel here")


# speedup vs baseline: 2.7869x; 2.7869x over previous
import functools

import jax
import jax.numpy as jnp
from jax.experimental import pallas as pl
from jax.experimental.pallas import tpu as pltpu

MAX_LEN = 4096
D_MODEL = 768
BATCH = 4
BL = 512
NB = MAX_LEN // BL
COND_PAD = 8


def _dense_body(in_ref, idx_ref, pos_ref, ctab_ref, out_ref):
    idx = idx_ref[0, 0, :]
    onehot = (idx[:, None] == jax.lax.broadcasted_iota(jnp.int32, (1, COND_PAD), 1)
              ).astype(jnp.float32)
    cond_emb = jnp.dot(onehot, ctab_ref[...],
                       preferred_element_type=jnp.float32)
    add = pos_ref[...] + cond_emb
    out_ref[...] = in_ref[...] + add[None, :, :]


@jax.jit
def _dense_add(inputs, cond_pos3, pos_table, cond_tab_padded):
    return pl.pallas_call(
        _dense_body,
        grid=(NB,),
        in_specs=[
            pl.BlockSpec((BATCH, BL, D_MODEL), lambda i: (0, i, 0)),
            pl.BlockSpec((1, 1, BL), lambda i: (i, 0, 0)),
            pl.BlockSpec((BL, D_MODEL), lambda i: (i, 0)),
            pl.BlockSpec((COND_PAD, D_MODEL), lambda i: (0, 0)),
        ],
        out_specs=pl.BlockSpec((BATCH, BL, D_MODEL), lambda i: (0, i, 0)),
        out_shape=jax.ShapeDtypeStruct((BATCH, MAX_LEN, D_MODEL), jnp.float32),
    )(inputs, cond_pos3, pos_table, cond_tab_padded)


def kernel(inputs, cond_pos, pos_table, cond_table):
    cond_pos3 = cond_pos.reshape(NB, 1, BL)
    ctab = jnp.zeros((COND_PAD, D_MODEL), jnp.float32).at[:cond_table.shape[0]].set(cond_table)
    return _dense_add(inputs, cond_pos3, pos_table, ctab)
